# baseline (speedup 1.0000x reference)
import dataclasses
import functools

import jax
import jax.numpy as jnp
from jax import lax
from jax.experimental import pallas as pl
from jax.experimental.pallas import tpu as pltpu
from jax.experimental.pallas import tpu_sc as plsc

_NUM_CORES = 2
_NUM_SUBCORES = 16
_LANES = 16
_NW = _NUM_CORES * _NUM_SUBCORES
_TBL_PAD = 128


def _sc_compiler_params():
    cp = pltpu.CompilerParams()
    if "needs_layout_passes" in pltpu.CompilerParams.__dataclass_fields__:
        cp = dataclasses.replace(cp, needs_layout_passes=False)
    return cp


def _embed_sc(tbl, inputs):
    b, l = inputs.shape
    rows = b // _NW
    offs = list(range(0, l - _LANES + 1, _LANES))
    if offs[-1] != l - _LANES:
        offs.append(l - _LANES)

    mesh = plsc.VectorSubcoreMesh(
        core_axis_name="c", subcore_axis_name="s",
        num_cores=_NUM_CORES, num_subcores=_NUM_SUBCORES,
    )

    blk_rows = 8

    @functools.partial(
        pl.kernel,
        out_type=jax.ShapeDtypeStruct((b, l), jnp.float32),
        mesh=mesh,
        scratch_types=[
            pltpu.VMEM((_TBL_PAD,), jnp.float32),
        ],
        compiler_params=_sc_compiler_params(),
    )
    def body(tbl_hbm, idx_hbm, out_hbm, tbl_v):
        pltpu.sync_copy(tbl_hbm, tbl_v)

        def block_body(idx_v, out_v):
            @plsc.parallel_loop(0, blk_rows, step=1, unroll=2)
            def _(r):
                for c in offs:
                    idx = idx_v[r, pl.ds(c, _LANES)]
                    out_v[r, pl.ds(c, _LANES)] = plsc.load_gather(tbl_v, [idx])

        pltpu.emit_pipeline(
            block_body,
            grid=(b // blk_rows,),
            in_specs=[pl.BlockSpec((blk_rows, l), lambda i: (i, 0))],
            out_specs=[pl.BlockSpec((blk_rows, l), lambda i: (i, 0))],
            core_axis_name=("c", "s"),
            dimension_semantics=(pltpu.PARALLEL,),
        )(idx_hbm, out_hbm)

    return body(tbl, inputs)


def _gather_tc(tbl, inputs, tc_rows):
    _, l = inputs.shape
    blk = 1024

    def body(idx_ref, tbl_ref, out_ref):
        idx = jnp.clip(idx_ref[...], 0, _TBL_PAD - 1)
        t = jnp.broadcast_to(tbl_ref[...][None, :], (blk, _TBL_PAD))
        out_ref[...] = jnp.take_along_axis(t, idx, axis=1, mode="promise_in_bounds")

    return pl.pallas_call(
        body,
        grid=(tc_rows // blk,),
        in_specs=[
            pl.BlockSpec((blk, l), lambda i: (i, 0)),
            pl.BlockSpec((_TBL_PAD,), lambda i: (0,)),
        ],
        out_specs=pl.BlockSpec((blk, l), lambda i: (i, 0)),
        out_shape=jax.ShapeDtypeStruct((tc_rows, l), jnp.float32),
    )(inputs, tbl)


_TC_ROWS = 3072


def kernel(inputs, z_weights):
    b, l = inputs.shape
    tbl = jnp.pad(z_weights[:, 0], (0, _TBL_PAD - z_weights.shape[0]))
    idx = inputs.astype(jnp.int32)
    sc_in = lax.slice(idx, (_TC_ROWS, 0), (b, l))
    tc_in = lax.slice(idx, (0, 0), (_TC_ROWS, l))
    out_sc = _embed_sc(tbl, sc_in)
    out_tc = _gather_tc(tbl, tc_in, _TC_ROWS)
    return jnp.concatenate([out_tc, out_sc], axis=0)[..., None]


# speedup vs baseline: 1.0432x; 1.0432x over previous
import dataclasses
import functools

import jax
import jax.numpy as jnp
from jax import lax
from jax.experimental import pallas as pl
from jax.experimental.pallas import tpu as pltpu
from jax.experimental.pallas import tpu_sc as plsc

_NUM_CORES = 2
_NUM_SUBCORES = 16
_LANES = 16
_NW = _NUM_CORES * _NUM_SUBCORES
_TBL_PAD = 128


def _sc_compiler_params():
    cp = pltpu.CompilerParams()
    if "needs_layout_passes" in pltpu.CompilerParams.__dataclass_fields__:
        cp = dataclasses.replace(cp, needs_layout_passes=False)
    return cp


def _embed_sc(tbl, inputs):
    b, l = inputs.shape
    rows = b // _NW
    offs = list(range(0, l - _LANES + 1, _LANES))
    if offs[-1] != l - _LANES:
        offs.append(l - _LANES)

    mesh = plsc.VectorSubcoreMesh(
        core_axis_name="c", subcore_axis_name="s",
        num_cores=_NUM_CORES, num_subcores=_NUM_SUBCORES,
    )

    blk_rows = 8

    @functools.partial(
        pl.kernel,
        out_type=jax.ShapeDtypeStruct((b, l), jnp.float32),
        mesh=mesh,
        scratch_types=[
            pltpu.VMEM((_TBL_PAD,), jnp.float32),
        ],
        compiler_params=_sc_compiler_params(),
    )
    def body(tbl_hbm, idx_hbm, out_hbm, tbl_v):
        pltpu.sync_copy(tbl_hbm, tbl_v)

        def block_body(idx_v, out_v):
            @plsc.parallel_loop(0, blk_rows, step=1, unroll=2)
            def _(r):
                for c in offs:
                    idx = idx_v[r, pl.ds(c, _LANES)]
                    out_v[r, pl.ds(c, _LANES)] = plsc.load_gather(tbl_v, [idx])

        pltpu.emit_pipeline(
            block_body,
            grid=(b // blk_rows,),
            in_specs=[pl.BlockSpec((blk_rows, l), lambda i: (i, 0))],
            out_specs=[pl.BlockSpec((blk_rows, l), lambda i: (i, 0))],
            core_axis_name=("c", "s"),
            dimension_semantics=(pltpu.PARALLEL,),
        )(idx_hbm, out_hbm)

    return body(tbl, inputs)


def _gather_tc(tbl, inputs, tc_rows):
    _, l = inputs.shape
    blk = 1024

    def body(idx_ref, tbl_ref, out_ref):
        idx = jnp.clip(idx_ref[...], 0, _TBL_PAD - 1)
        t = jnp.broadcast_to(tbl_ref[...][None, :], (blk, _TBL_PAD))
        out_ref[...] = jnp.take_along_axis(t, idx, axis=1, mode="promise_in_bounds")

    return pl.pallas_call(
        body,
        grid=(tc_rows // blk,),
        in_specs=[
            pl.BlockSpec((blk, l), lambda i: (i, 0)),
            pl.BlockSpec((_TBL_PAD,), lambda i: (0,)),
        ],
        out_specs=pl.BlockSpec((blk, l), lambda i: (i, 0)),
        out_shape=jax.ShapeDtypeStruct((tc_rows, l), jnp.float32),
    )(inputs, tbl)


_TC_ROWS = 2816


def kernel(inputs, z_weights):
    b, l = inputs.shape
    tbl = jnp.pad(z_weights[:, 0], (0, _TBL_PAD - z_weights.shape[0]))
    idx = inputs.astype(jnp.int32)
    sc_in = lax.slice(idx, (_TC_ROWS, 0), (b, l))
    tc_in = lax.slice(idx, (0, 0), (_TC_ROWS, l))
    out_sc = _embed_sc(tbl, sc_in)
    out_tc = _gather_tc(tbl, tc_in, _TC_ROWS)
    return jnp.concatenate([out_tc, out_sc], axis=0)[..., None]
